# baseline (speedup 1.0000x reference)
import functools

import jax
import jax.numpy as jnp
from jax import lax
from jax.experimental import pallas as pl
from jax.experimental.pallas import tpu as pltpu
from jax.experimental.pallas import tpu_sc as plsc

VOCAB = 1000000
EMBED_DIM = 64
SLOT = 128
SEQ = 200
ABLK = 512
N_FULL_BLOCKS = VOCAB // ABLK
TAIL_ROWS = VOCAB - N_FULL_BLOCKS * ABLK
B_PER_CHUNK = 1
NBUF = 2

_params = pltpu.CompilerParams(needs_layout_passes=False)


@jax.jit
def _sc_embed(weight_t, wtail_t, idx_flat):
    info = plsc.get_sparse_core_info()
    nc, ns = info.num_cores, info.num_subcores
    nw = nc * ns
    mesh = plsc.VectorSubcoreMesh(core_axis_name="c", subcore_axis_name="s")

    @functools.partial(
        pl.kernel,
        mesh=mesh,
        out_type=jax.ShapeDtypeStruct((VOCAB, SLOT), jnp.float32),
        scratch_types=[
            pltpu.VMEM((EMBED_DIM, ABLK), jnp.float32),
            pltpu.VMEM((ABLK, SLOT), jnp.float32),
            pltpu.VMEM((EMBED_DIM, TAIL_ROWS), jnp.float32),
            pltpu.VMEM((TAIL_ROWS, SLOT), jnp.float32),
        ],
        compiler_params=_params,
    )
    def transpose_k(wt_hbm, wtail_hbm, tab_hbm, in_v, tr_v, in_t, tr_t):
        wid = lax.axis_index("s") * nc + lax.axis_index("c")

        def do_block(src_ref, dst_ref, n_rows):
            def rbody(rg, carry):
                for u in range(8):
                    r = rg * 8 + u
                    for g in range(EMBED_DIM // 16):
                        vals = plsc.load_gather(
                            src_ref,
                            [lax.iota(jnp.int32, 16) + g * 16,
                             jnp.full((16,), r, jnp.int32)])
                        dst_ref[r, pl.ds(g * 16, 16)] = vals
                return carry

            lax.fori_loop(0, n_rows // 8, rbody, 0)

        n_iters = (N_FULL_BLOCKS + nw - 1) // nw

        def body(k_it, carry):
            blk = wid + k_it * nw

            @pl.when(blk < N_FULL_BLOCKS)
            def _():
                off = pl.multiple_of(blk * ABLK, SLOT)
                pltpu.sync_copy(wt_hbm.at[:, pl.ds(off, ABLK)], in_v)
                do_block(in_v, tr_v, ABLK)
                pltpu.sync_copy(tr_v, tab_hbm.at[pl.ds(off, ABLK)])
            return carry

        lax.fori_loop(0, n_iters, body, 0)

        @pl.when(wid == nw - 1)
        def _():
            pltpu.sync_copy(wtail_hbm, in_t)
            do_block(in_t, tr_t, TAIL_ROWS)
            pltpu.sync_copy(
                tr_t, tab_hbm.at[pl.ds(N_FULL_BLOCKS * ABLK, TAIL_ROWS)])

    table = transpose_k(weight_t, wtail_t)

    n_batch = idx_flat.shape[0] // SEQ
    b_per_w = n_batch // nw
    n_steps = b_per_w // B_PER_CHUNK
    n_outer = n_steps // NBUF
    chunk_idx = B_PER_CHUNK * SEQ

    @functools.partial(
        pl.kernel,
        mesh=mesh,
        out_type=jax.ShapeDtypeStruct((n_batch, SEQ, EMBED_DIM), jnp.float32),
        scratch_types=[
            pltpu.VMEM((SEQ,), jnp.int32),
            pltpu.VMEM((SEQ,), jnp.int32),
            pltpu.VMEM((B_PER_CHUNK, SEQ, SLOT), jnp.float32),
            pltpu.VMEM((NBUF, B_PER_CHUNK, SEQ, EMBED_DIM), jnp.float32),
            pltpu.SemaphoreType.DMA,
            pltpu.SemaphoreType.DMA,
            pltpu.SemaphoreType.DMA,
            pltpu.SemaphoreType.DMA,
            pltpu.SemaphoreType.DMA,
        ],
        compiler_params=_params,
    )
    def gather_k(tab_hbm, idx_hbm, out_hbm, idx_v00, idx_v10,
                 rows_v, rows64_v, sem_i0, sem_i1, sem_g,
                 sem_s0, sem_s1):
        idx_bufs = ((idx_v00,), (idx_v10,))
        wid = lax.axis_index("s") * nc + lax.axis_index("c")
        b_base = wid * b_per_w
        sem_i = (sem_i0, sem_i1)
        sem_s = (sem_s0, sem_s1)

        def idx_copies(step, buf):
            cs = []
            for j in range(B_PER_CHUNK):
                off = pl.multiple_of(
                    (b_base + step * B_PER_CHUNK + j) * SEQ, 8)
                cs.append(pltpu.make_async_copy(
                    idx_hbm.at[pl.ds(off, SEQ)], idx_bufs[buf][j],
                    sem_i[buf]))
            return cs

        def store_copy(step, buf):
            b_off = b_base + step * B_PER_CHUNK
            return pltpu.make_async_copy(
                rows64_v.at[buf], out_hbm.at[pl.ds(b_off, B_PER_CHUNK)],
                sem_s[buf])

        for c in idx_copies(0, 0):
            c.start()
        for c in idx_copies(1, 1):
            c.start()

        def outer(g, carry):
            for b in range(NBUF):
                step = g * NBUF + b
                for c in idx_copies(step, b):
                    c.wait()

                handles = []
                for j in range(B_PER_CHUNK):
                    handles.append(
                        pltpu.async_copy(
                            tab_hbm.at[idx_bufs[b][j]],
                            rows_v.at[j], sem_g))
                for h in handles:
                    h.wait()

                @pl.when(g > 0)
                def _():
                    store_copy(step - NBUF, b).wait()

                def cbody(rg, carry2):
                    for u in range(4):
                        r = rg * 4 + u
                        for j in range(B_PER_CHUNK):
                            for gg in range(EMBED_DIM // 16):
                                rows64_v[b, j, r, pl.ds(gg * 16, 16)] = (
                                    rows_v[j, r, pl.ds(gg * 16, 16)])
                    return carry2

                lax.fori_loop(0, SEQ // 4, cbody, 0)

                store_copy(step, b).start()

                @pl.when(g < n_outer - 1)
                def _():
                    for c in idx_copies(step + NBUF, b):
                        c.start()
            return carry

        lax.fori_loop(0, n_outer, outer, 0)

        for b in range(NBUF):
            store_copy(n_steps - NBUF + b, b).wait()

    return gather_k(table, idx_flat)


def kernel(input_, weight):
    b, s = input_.shape
    idx_flat = input_.reshape(b * s).astype(jnp.int32)
    tail = weight[N_FULL_BLOCKS * ABLK:].T
    return _sc_embed(weight.T, tail, idx_flat)


# speedup vs baseline: 2.0708x; 2.0708x over previous
import functools

import jax
import jax.numpy as jnp
from jax import lax
from jax.experimental import pallas as pl
from jax.experimental.pallas import tpu as pltpu
from jax.experimental.pallas import tpu_sc as plsc

VOCAB = 1000000
EMBED_DIM = 64
SLOT = 128
SEQ = 200
ABLK = 512
N_FULL_BLOCKS = VOCAB // ABLK
TAIL_ROWS = VOCAB - N_FULL_BLOCKS * ABLK
B_PER_CHUNK = 1
NBUF = 2

_params = pltpu.CompilerParams(needs_layout_passes=False)


@jax.jit
def _sc_embed(table128, idx_flat):
    info = plsc.get_sparse_core_info()
    nc, ns = info.num_cores, info.num_subcores
    nw = nc * ns
    mesh = plsc.VectorSubcoreMesh(core_axis_name="c", subcore_axis_name="s")

    @functools.partial(
        pl.kernel,
        mesh=mesh,
        out_type=jax.ShapeDtypeStruct((VOCAB, SLOT), jnp.float32),
        scratch_types=[
            pltpu.VMEM((EMBED_DIM, ABLK), jnp.float32),
            pltpu.VMEM((ABLK, SLOT), jnp.float32),
            pltpu.VMEM((EMBED_DIM, TAIL_ROWS), jnp.float32),
            pltpu.VMEM((TAIL_ROWS, SLOT), jnp.float32),
        ],
        compiler_params=_params,
    )
    def transpose_k(wt_hbm, wtail_hbm, tab_hbm, in_v, tr_v, in_t, tr_t):
        wid = lax.axis_index("s") * nc + lax.axis_index("c")

        def do_block(src_ref, dst_ref, n_rows):
            def rbody(rg, carry):
                for u in range(8):
                    r = rg * 8 + u
                    for g in range(EMBED_DIM // 16):
                        vals = plsc.load_gather(
                            src_ref,
                            [lax.iota(jnp.int32, 16) + g * 16,
                             jnp.full((16,), r, jnp.int32)])
                        dst_ref[r, pl.ds(g * 16, 16)] = vals
                return carry

            lax.fori_loop(0, n_rows // 8, rbody, 0)

        n_iters = (N_FULL_BLOCKS + nw - 1) // nw

        def body(k_it, carry):
            blk = wid + k_it * nw

            @pl.when(blk < N_FULL_BLOCKS)
            def _():
                off = pl.multiple_of(blk * ABLK, SLOT)
                pltpu.sync_copy(wt_hbm.at[:, pl.ds(off, ABLK)], in_v)
                do_block(in_v, tr_v, ABLK)
                pltpu.sync_copy(tr_v, tab_hbm.at[pl.ds(off, ABLK)])
            return carry

        lax.fori_loop(0, n_iters, body, 0)

        @pl.when(wid == nw - 1)
        def _():
            pltpu.sync_copy(wtail_hbm, in_t)
            do_block(in_t, tr_t, TAIL_ROWS)
            pltpu.sync_copy(
                tr_t, tab_hbm.at[pl.ds(N_FULL_BLOCKS * ABLK, TAIL_ROWS)])

    table = table128

    n_batch = idx_flat.shape[0] // SEQ
    b_per_w = n_batch // nw
    n_steps = b_per_w // B_PER_CHUNK
    n_outer = n_steps // NBUF
    chunk_idx = B_PER_CHUNK * SEQ

    @functools.partial(
        pl.kernel,
        mesh=mesh,
        out_type=jax.ShapeDtypeStruct((n_batch, SEQ, EMBED_DIM), jnp.float32),
        scratch_types=[
            pltpu.VMEM((SEQ,), jnp.int32),
            pltpu.VMEM((SEQ,), jnp.int32),
            pltpu.VMEM((B_PER_CHUNK, SEQ, SLOT), jnp.float32),
            pltpu.VMEM((NBUF, B_PER_CHUNK, SEQ, EMBED_DIM), jnp.float32),
            pltpu.SemaphoreType.DMA,
            pltpu.SemaphoreType.DMA,
            pltpu.SemaphoreType.DMA,
            pltpu.SemaphoreType.DMA,
            pltpu.SemaphoreType.DMA,
        ],
        compiler_params=_params,
    )
    def gather_k(tab_hbm, idx_hbm, out_hbm, idx_v00, idx_v10,
                 rows_v, rows64_v, sem_i0, sem_i1, sem_g,
                 sem_s0, sem_s1):
        idx_bufs = ((idx_v00,), (idx_v10,))
        wid = lax.axis_index("s") * nc + lax.axis_index("c")
        b_base = wid * b_per_w
        sem_i = (sem_i0, sem_i1)
        sem_s = (sem_s0, sem_s1)

        def idx_copies(step, buf):
            cs = []
            for j in range(B_PER_CHUNK):
                off = pl.multiple_of(
                    (b_base + step * B_PER_CHUNK + j) * SEQ, 8)
                cs.append(pltpu.make_async_copy(
                    idx_hbm.at[pl.ds(off, SEQ)], idx_bufs[buf][j],
                    sem_i[buf]))
            return cs

        def store_copy(step, buf):
            b_off = b_base + step * B_PER_CHUNK
            return pltpu.make_async_copy(
                rows64_v.at[buf], out_hbm.at[pl.ds(b_off, B_PER_CHUNK)],
                sem_s[buf])

        for c in idx_copies(0, 0):
            c.start()
        for c in idx_copies(1, 1):
            c.start()

        def outer(g, carry):
            for b in range(NBUF):
                step = g * NBUF + b
                for c in idx_copies(step, b):
                    c.wait()

                handles = []
                for j in range(B_PER_CHUNK):
                    handles.append(
                        pltpu.async_copy(
                            tab_hbm.at[idx_bufs[b][j]],
                            rows_v.at[j], sem_g))
                for h in handles:
                    h.wait()

                @pl.when(g > 0)
                def _():
                    store_copy(step - NBUF, b).wait()

                def cbody(rg, carry2):
                    for u in range(4):
                        r = rg * 4 + u
                        for j in range(B_PER_CHUNK):
                            for gg in range(EMBED_DIM // 16):
                                rows64_v[b, j, r, pl.ds(gg * 16, 16)] = (
                                    rows_v[j, r, pl.ds(gg * 16, 16)])
                    return carry2

                lax.fori_loop(0, SEQ // 4, cbody, 0)

                store_copy(step, b).start()

                @pl.when(g < n_outer - 1)
                def _():
                    for c in idx_copies(step + NBUF, b):
                        c.start()
            return carry

        lax.fori_loop(0, n_outer, outer, 0)

        for b in range(NBUF):
            store_copy(n_steps - NBUF + b, b).wait()

    return gather_k(table, idx_flat)


def kernel(input_, weight):
    b, s = input_.shape
    idx_flat = input_.reshape(b * s).astype(jnp.int32)
    table128 = jnp.pad(weight, ((0, 0), (0, SLOT - EMBED_DIM)))
    return _sc_embed(table128, idx_flat)


# speedup vs baseline: 2.7014x; 1.3045x over previous
import functools

import jax
import jax.numpy as jnp
from jax import lax
from jax.experimental import pallas as pl
from jax.experimental.pallas import tpu as pltpu
from jax.experimental.pallas import tpu_sc as plsc

VOCAB = 1000000
EMBED_DIM = 64
SLOT = 128
SEQ = 200
ABLK = 512
N_FULL_BLOCKS = VOCAB // ABLK
TAIL_ROWS = VOCAB - N_FULL_BLOCKS * ABLK
B_PER_CHUNK = 1
NBUF = 2

_params = pltpu.CompilerParams(needs_layout_passes=False)


@jax.jit
def _sc_embed(table128, idx_flat):
    info = plsc.get_sparse_core_info()
    nc, ns = info.num_cores, info.num_subcores
    nw = nc * ns
    mesh = plsc.VectorSubcoreMesh(core_axis_name="c", subcore_axis_name="s")

    @functools.partial(
        pl.kernel,
        mesh=mesh,
        out_type=jax.ShapeDtypeStruct((VOCAB, SLOT), jnp.float32),
        scratch_types=[
            pltpu.VMEM((EMBED_DIM, ABLK), jnp.float32),
            pltpu.VMEM((ABLK, SLOT), jnp.float32),
            pltpu.VMEM((EMBED_DIM, TAIL_ROWS), jnp.float32),
            pltpu.VMEM((TAIL_ROWS, SLOT), jnp.float32),
        ],
        compiler_params=_params,
    )
    def transpose_k(wt_hbm, wtail_hbm, tab_hbm, in_v, tr_v, in_t, tr_t):
        wid = lax.axis_index("s") * nc + lax.axis_index("c")

        def do_block(src_ref, dst_ref, n_rows):
            def rbody(rg, carry):
                for u in range(8):
                    r = rg * 8 + u
                    for g in range(EMBED_DIM // 16):
                        vals = plsc.load_gather(
                            src_ref,
                            [lax.iota(jnp.int32, 16) + g * 16,
                             jnp.full((16,), r, jnp.int32)])
                        dst_ref[r, pl.ds(g * 16, 16)] = vals
                return carry

            lax.fori_loop(0, n_rows // 8, rbody, 0)

        n_iters = (N_FULL_BLOCKS + nw - 1) // nw

        def body(k_it, carry):
            blk = wid + k_it * nw

            @pl.when(blk < N_FULL_BLOCKS)
            def _():
                off = pl.multiple_of(blk * ABLK, SLOT)
                pltpu.sync_copy(wt_hbm.at[:, pl.ds(off, ABLK)], in_v)
                do_block(in_v, tr_v, ABLK)
                pltpu.sync_copy(tr_v, tab_hbm.at[pl.ds(off, ABLK)])
            return carry

        lax.fori_loop(0, n_iters, body, 0)

        @pl.when(wid == nw - 1)
        def _():
            pltpu.sync_copy(wtail_hbm, in_t)
            do_block(in_t, tr_t, TAIL_ROWS)
            pltpu.sync_copy(
                tr_t, tab_hbm.at[pl.ds(N_FULL_BLOCKS * ABLK, TAIL_ROWS)])

    table = table128

    n_batch = idx_flat.shape[0] // SEQ
    b_per_w = n_batch // nw
    n_steps = b_per_w // B_PER_CHUNK
    n_outer = n_steps // NBUF
    chunk_idx = B_PER_CHUNK * SEQ

    @functools.partial(
        pl.kernel,
        mesh=mesh,
        out_type=jax.ShapeDtypeStruct((n_batch, SEQ, EMBED_DIM), jnp.float32),
        scratch_types=[
            pltpu.VMEM((SEQ,), jnp.int32),
            pltpu.VMEM((SEQ,), jnp.int32),
            pltpu.VMEM((NBUF, SEQ, SLOT), jnp.float32),
            pltpu.VMEM((NBUF, 1, SEQ, EMBED_DIM), jnp.float32),
            pltpu.SemaphoreType.DMA,
            pltpu.SemaphoreType.DMA,
            pltpu.SemaphoreType.DMA,
            pltpu.SemaphoreType.DMA,
            pltpu.SemaphoreType.DMA,
            pltpu.SemaphoreType.DMA,
        ],
        compiler_params=_params,
    )
    def gather_k(tab_hbm, idx_hbm, out_hbm, idx_v0, idx_v1, rows_v,
                 rows64_v, sem_i0, sem_i1, sem_g0, sem_g1, sem_s0, sem_s1):
        idx_bufs = (idx_v0, idx_v1)
        wid = lax.axis_index("s") * nc + lax.axis_index("c")
        b_base = wid * b_per_w
        sem_i = (sem_i0, sem_i1)
        sem_g = (sem_g0, sem_g1)
        sem_s = (sem_s0, sem_s1)

        def idx_copy(step, buf):
            off = pl.multiple_of((b_base + step) * SEQ, 8)
            return pltpu.make_async_copy(
                idx_hbm.at[pl.ds(off, SEQ)], idx_bufs[buf], sem_i[buf])

        def gather_copy(buf):
            return pltpu.make_async_copy(
                tab_hbm.at[idx_bufs[buf]], rows_v.at[buf], sem_g[buf])

        def store_copy(step, buf):
            b_off = b_base + step
            return pltpu.make_async_copy(
                rows64_v.at[buf], out_hbm.at[pl.ds(b_off, 1)], sem_s[buf])

        def compact(buf):
            def cbody(rg, carry2):
                for u in range(4):
                    r = rg * 4 + u
                    for gg in range(EMBED_DIM // 16):
                        rows64_v[buf, 0, r, pl.ds(gg * 16, 16)] = (
                            rows_v[buf, r, pl.ds(gg * 16, 16)])
                return carry2

            lax.fori_loop(0, SEQ // 4, cbody, 0)

        idx_copy(0, 0).start()
        idx_copy(1, 1).start()

        def outer(g, carry):
            for b in range(NBUF):
                step = g * NBUF + b
                pb = 1 - b
                idx_copy(step, b).wait()
                gather_copy(b).start()

                @pl.when(step > 0)
                def _():
                    gather_copy(pb).wait()

                    @pl.when(step > 2)
                    def _():
                        store_copy(step - 3, pb).wait()

                    compact(pb)
                    store_copy(step - 1, pb).start()

                    @pl.when(step + 1 < n_steps)
                    def _():
                        idx_copy(step + 1, pb).start()
            return carry

        lax.fori_loop(0, n_outer, outer, 0)

        last = n_steps - 1
        lb = last % NBUF
        gather_copy(lb).wait()
        store_copy(last - 2, lb).wait()
        compact(lb)
        store_copy(last, lb).start()
        store_copy(last - 1, 1 - lb).wait()
        store_copy(last, lb).wait()

    return gather_k(table, idx_flat)


def kernel(input_, weight):
    b, s = input_.shape
    idx_flat = input_.reshape(b * s).astype(jnp.int32)
    table128 = jnp.pad(weight, ((0, 0), (0, SLOT - EMBED_DIM)))
    return _sc_embed(table128, idx_flat)
